# baseline (speedup 1.0000x reference)
import functools

import jax
import jax.numpy as jnp
from jax import lax
from jax.experimental import pallas as pl
from jax.experimental.pallas import tpu as pltpu
from jax.experimental.pallas import tpu_sc as plsc

N = 10000
D = 128
H = 128

_NC = 2
_NS = 16
_NW = _NC * _NS

_C = 200
_BE = 8000

_MESH_SEGS = (32000, 64000, 64000, 64000, 96000)
_WORLD_SEGS = (64000, 64000, 32000)


def _proj_body(nf_ref, ws_ref, wd_ref, ps_ref, pd_ref):
    x = nf_ref[...]
    ps_ref[...] = jnp.dot(x, ws_ref[...], preferred_element_type=jnp.float32)
    pd_ref[...] = jnp.dot(x, wd_ref[...], preferred_element_type=jnp.float32)


def _node_proj(nfeat, w1s, w1d):
    BN = 2000
    return pl.pallas_call(
        _proj_body,
        grid=(N // BN,),
        in_specs=[
            pl.BlockSpec((BN, D), lambda i: (i, 0)),
            pl.BlockSpec((D, H), lambda i: (0, 0)),
            pl.BlockSpec((D, H), lambda i: (0, 0)),
        ],
        out_specs=[
            pl.BlockSpec((BN, H), lambda i: (i, 0)),
            pl.BlockSpec((BN, H), lambda i: (i, 0)),
        ],
        out_shape=[
            jax.ShapeDtypeStruct((N, H), jnp.float32),
            jax.ShapeDtypeStruct((N, H), jnp.float32),
        ],
    )(nfeat, w1s, w1d)


def _gather_add(ps, pd, eidx_flat, E, seg_base, seg_e):
    EPW = seg_e // _NW
    C = min(_C, EPW)
    nchunks = EPW // C
    npairs = nchunks // 2
    has_tail = (nchunks % 2) == 1
    mesh = plsc.VectorSubcoreMesh(core_axis_name="c", subcore_axis_name="s")

    @functools.partial(
        pl.kernel,
        mesh=mesh,
        out_type=jax.ShapeDtypeStruct((seg_e, H), jnp.float32),
        scratch_types=[
            pltpu.VMEM((EPW,), jnp.int32),
            pltpu.VMEM((EPW,), jnp.int32),
            pltpu.VMEM((2, C, H), jnp.float32),
            pltpu.VMEM((2, C, H), jnp.float32),
            pltpu.SemaphoreType.DMA,
            pltpu.SemaphoreType.DMA,
            pltpu.SemaphoreType.DMA,
            pltpu.SemaphoreType.DMA,
        ],
    )
    def k(ps_hbm, pd_hbm, ei_hbm, out_hbm, si_v, di_v, rs_v, rd_v,
          g0, g1, w0, w1):
        cid = lax.axis_index("c")
        sid = lax.axis_index("s")
        wid = sid * _NC + cid
        wbase = wid * EPW
        gsem = (g0, g1)
        wsem = (w0, w1)

        pltpu.sync_copy(ei_hbm.at[pl.ds(seg_base + wbase, EPW)], si_v)
        pltpu.sync_copy(ei_hbm.at[pl.ds(E + seg_base + wbase, EPW)], di_v)

        def issue_gather(ci, b):
            off = ci * C
            pltpu.async_copy(ps_hbm.at[si_v.at[pl.ds(off, C)]], rs_v.at[b],
                             gsem[b])
            pltpu.async_copy(pd_hbm.at[di_v.at[pl.ds(off, C)]], rd_v.at[b],
                             gsem[b])

        def wait_gather(ci, b):
            off = ci * C
            pltpu.make_async_copy(ps_hbm.at[si_v.at[pl.ds(off, C)]],
                                  rs_v.at[b], gsem[b]).wait()
            pltpu.make_async_copy(pd_hbm.at[di_v.at[pl.ds(off, C)]],
                                  rd_v.at[b], gsem[b]).wait()

        def add_rows(b):
            def row(r, c2):
                for g in range(H // 16):
                    sl = pl.ds(g * 16, 16)
                    rd_v[b, r, sl] = rs_v[b, r, sl] + rd_v[b, r, sl]
                return c2
            lax.fori_loop(0, C, row, 0)

        def issue_write(ci, b):
            sl = pl.ds(wbase + ci * C, C)
            pltpu.async_copy(rd_v.at[b], out_hbm.at[sl], wsem[b])

        def wait_write(ci, b):
            sl = pl.ds(wbase + ci * C, C)
            pltpu.make_async_copy(rd_v.at[b], out_hbm.at[sl],
                                  wsem[b]).wait()

        issue_gather(0, 0)
        if nchunks > 1:
            issue_gather(1, 1)

        def pair(p, carry):
            i0 = 2 * p
            wait_gather(i0, 0)
            add_rows(0)
            issue_write(i0, 0)
            wait_gather(i0 + 1, 1)
            add_rows(1)
            issue_write(i0 + 1, 1)

            @pl.when(p + 1 < npairs + (1 if has_tail else 0))
            def _():
                wait_write(i0, 0)
                issue_gather(i0 + 2, 0)

                @pl.when(p + 1 < npairs)
                def _():
                    wait_write(i0 + 1, 1)
                    issue_gather(i0 + 3, 1)

            return carry

        lax.fori_loop(0, npairs, pair, 0)

        last = nchunks - 1
        if has_tail:
            wait_gather(last, 0)
            add_rows(0)
            issue_write(last, 0)
            wait_write(last, 0)
            if npairs > 0:
                wait_write(last - 1, 1)
        else:
            wait_write(last - 1, 0)
            wait_write(last, 1)

    return k(ps, pd, eidx_flat)


def _mlp_compute(ef, acc, w1e_ref, w2_ref, b1_ref, b2_ref, g_ref, bb_ref):
    h = (jnp.dot(ef, w1e_ref[...], preferred_element_type=jnp.float32)
         + acc + b1_ref[...])
    h = h * jax.nn.sigmoid(h)
    o = jnp.dot(h, w2_ref[...], preferred_element_type=jnp.float32) + b2_ref[...]
    mu = jnp.mean(o, axis=-1, keepdims=True)
    var = jnp.mean((o - mu) ** 2, axis=-1, keepdims=True)
    o = g_ref[...] * (o - mu) * lax.rsqrt(var + 1e-5) + bb_ref[...]
    return o + ef


def _mlp_first_body(ef_ref, acc_ref, w1e_ref, w2_ref, b1_ref, b2_ref, g_ref,
                    bb_ref, out_ref):
    out_ref[...] = _mlp_compute(ef_ref[...], acc_ref[...], w1e_ref, w2_ref,
                                b1_ref, b2_ref, g_ref, bb_ref)


def _mlp_chain_body(carry_ref, ef_ref, acc_ref, w1e_ref, w2_ref, b1_ref,
                    b2_ref, g_ref, bb_ref, out_ref):
    del carry_ref
    out_ref[...] = _mlp_compute(ef_ref[...], acc_ref[...], w1e_ref, w2_ref,
                                b1_ref, b2_ref, g_ref, bb_ref)


def _edge_mlp(ef, accs, w1e, w2, b1r, b2r, gr, br):
    E = ef.shape[0]
    vspec = pl.BlockSpec((1, H), lambda i: (0, 0))
    wspec = [
        pl.BlockSpec((D, H), lambda i: (0, 0)),
        pl.BlockSpec((H, D), lambda i: (0, 0)),
        vspec, vspec, vspec, vspec,
    ]
    out = None
    for acc, base, seg_e in accs:
        nblk = seg_e // _BE
        bb = base // _BE
        ef_spec = pl.BlockSpec((_BE, D), lambda i, bb=bb: (bb + i, 0))
        acc_spec = pl.BlockSpec((_BE, H), lambda i: (i, 0))
        out_spec = pl.BlockSpec((_BE, D), lambda i, bb=bb: (bb + i, 0))
        if out is None:
            out = pl.pallas_call(
                _mlp_first_body,
                grid=(nblk,),
                in_specs=[ef_spec, acc_spec] + wspec,
                out_specs=out_spec,
                out_shape=jax.ShapeDtypeStruct((E, D), jnp.float32),
            )(ef, acc, w1e, w2, b1r, b2r, gr, br)
        else:
            out = pl.pallas_call(
                _mlp_chain_body,
                grid=(nblk,),
                in_specs=[pl.BlockSpec(memory_space=pl.ANY), ef_spec,
                          acc_spec] + wspec,
                out_specs=out_spec,
                out_shape=jax.ShapeDtypeStruct((E, D), jnp.float32),
                input_output_aliases={0: 0},
            )(out, ef, acc, w1e, w2, b1r, b2r, gr, br)
    return out


def kernel(mesh_efeat, world_efeat, nfeat, mesh_edge_index, world_edge_index,
           W1, b1, W2, b2, ln_g, ln_b):
    w1e = W1[:D]
    w1s = W1[D:2 * D]
    w1d = W1[2 * D:]

    ps, pd = _node_proj(nfeat, w1s, w1d)

    m_eidx = mesh_edge_index.reshape(-1)
    w_eidx = world_edge_index.reshape(-1)
    E_M = mesh_edge_index.shape[1]
    E_W = world_edge_index.shape[1]

    def build_segs(eidx, E, segs):
        assert sum(segs) == E
        entries, base = [], 0
        for s in segs:
            entries.append((base, s))
            base += s
        return [(_gather_add(ps, pd, eidx, E, b, s), b, s)
                for b, s in entries]

    acc_m = build_segs(m_eidx, E_M, _MESH_SEGS)
    acc_w = build_segs(w_eidx, E_W, _WORLD_SEGS)

    b1r = b1.reshape(1, H)
    b2r = b2.reshape(1, D)
    gr = ln_g.reshape(1, D)
    br = ln_b.reshape(1, D)

    mesh_new = _edge_mlp(mesh_efeat, acc_m, w1e, W2, b1r, b2r, gr, br)
    world_new = _edge_mlp(world_efeat, acc_w, w1e, W2, b1r, b2r, gr, br)
    return (mesh_new, world_new, nfeat)


# speedup vs baseline: 1.0060x; 1.0060x over previous
import functools

import jax
import jax.numpy as jnp
from jax import lax
from jax.experimental import pallas as pl
from jax.experimental.pallas import tpu as pltpu
from jax.experimental.pallas import tpu_sc as plsc

N = 10000
D = 128
H = 128

_NC = 2
_NS = 16
_NW = _NC * _NS

_C = 200
_BE = 4000

_MESH_SEGS = (32000, 64000, 64000, 64000, 96000)
_WORLD_SEGS = (64000, 64000, 32000)


def _proj_body(nf_ref, ws_ref, wd_ref, ps_ref, pd_ref):
    x = nf_ref[...]
    ps_ref[...] = jnp.dot(x, ws_ref[...], preferred_element_type=jnp.float32)
    pd_ref[...] = jnp.dot(x, wd_ref[...], preferred_element_type=jnp.float32)


def _node_proj(nfeat, w1s, w1d):
    BN = 2000
    return pl.pallas_call(
        _proj_body,
        grid=(N // BN,),
        in_specs=[
            pl.BlockSpec((BN, D), lambda i: (i, 0)),
            pl.BlockSpec((D, H), lambda i: (0, 0)),
            pl.BlockSpec((D, H), lambda i: (0, 0)),
        ],
        out_specs=[
            pl.BlockSpec((BN, H), lambda i: (i, 0)),
            pl.BlockSpec((BN, H), lambda i: (i, 0)),
        ],
        out_shape=[
            jax.ShapeDtypeStruct((N, H), jnp.float32),
            jax.ShapeDtypeStruct((N, H), jnp.float32),
        ],
    )(nfeat, w1s, w1d)


def _gather_add(ps, pd, eidx_flat, E, seg_base, seg_e):
    EPW = seg_e // _NW
    C = min(_C, EPW)
    nchunks = EPW // C
    npairs = nchunks // 2
    has_tail = (nchunks % 2) == 1
    mesh = plsc.VectorSubcoreMesh(core_axis_name="c", subcore_axis_name="s")

    @functools.partial(
        pl.kernel,
        mesh=mesh,
        out_type=jax.ShapeDtypeStruct((seg_e, H), jnp.float32),
        scratch_types=[
            pltpu.VMEM((EPW,), jnp.int32),
            pltpu.VMEM((EPW,), jnp.int32),
            pltpu.VMEM((2, C, H), jnp.float32),
            pltpu.VMEM((2, C, H), jnp.float32),
            pltpu.SemaphoreType.DMA,
            pltpu.SemaphoreType.DMA,
            pltpu.SemaphoreType.DMA,
            pltpu.SemaphoreType.DMA,
        ],
    )
    def k(ps_hbm, pd_hbm, ei_hbm, out_hbm, si_v, di_v, rs_v, rd_v,
          g0, g1, w0, w1):
        cid = lax.axis_index("c")
        sid = lax.axis_index("s")
        wid = sid * _NC + cid
        wbase = wid * EPW
        gsem = (g0, g1)
        wsem = (w0, w1)

        pltpu.sync_copy(ei_hbm.at[pl.ds(seg_base + wbase, EPW)], si_v)
        pltpu.sync_copy(ei_hbm.at[pl.ds(E + seg_base + wbase, EPW)], di_v)

        def issue_gather(ci, b):
            off = ci * C
            pltpu.async_copy(ps_hbm.at[si_v.at[pl.ds(off, C)]], rs_v.at[b],
                             gsem[b])
            pltpu.async_copy(pd_hbm.at[di_v.at[pl.ds(off, C)]], rd_v.at[b],
                             gsem[b])

        def wait_gather(ci, b):
            off = ci * C
            pltpu.make_async_copy(ps_hbm.at[si_v.at[pl.ds(off, C)]],
                                  rs_v.at[b], gsem[b]).wait()
            pltpu.make_async_copy(pd_hbm.at[di_v.at[pl.ds(off, C)]],
                                  rd_v.at[b], gsem[b]).wait()

        def add_rows(b):
            def row(r, c2):
                for g in range(H // 16):
                    sl = pl.ds(g * 16, 16)
                    rd_v[b, r, sl] = rs_v[b, r, sl] + rd_v[b, r, sl]
                return c2
            lax.fori_loop(0, C, row, 0)

        def issue_write(ci, b):
            sl = pl.ds(wbase + ci * C, C)
            pltpu.async_copy(rd_v.at[b], out_hbm.at[sl], wsem[b])

        def wait_write(ci, b):
            sl = pl.ds(wbase + ci * C, C)
            pltpu.make_async_copy(rd_v.at[b], out_hbm.at[sl],
                                  wsem[b]).wait()

        issue_gather(0, 0)
        if nchunks > 1:
            issue_gather(1, 1)

        def pair(p, carry):
            i0 = 2 * p
            wait_gather(i0, 0)
            add_rows(0)
            issue_write(i0, 0)
            wait_gather(i0 + 1, 1)
            add_rows(1)
            issue_write(i0 + 1, 1)

            @pl.when(p + 1 < npairs + (1 if has_tail else 0))
            def _():
                wait_write(i0, 0)
                issue_gather(i0 + 2, 0)

                @pl.when(p + 1 < npairs)
                def _():
                    wait_write(i0 + 1, 1)
                    issue_gather(i0 + 3, 1)

            return carry

        lax.fori_loop(0, npairs, pair, 0)

        last = nchunks - 1
        if has_tail:
            wait_gather(last, 0)
            add_rows(0)
            issue_write(last, 0)
            wait_write(last, 0)
            if npairs > 0:
                wait_write(last - 1, 1)
        else:
            wait_write(last - 1, 0)
            wait_write(last, 1)

    return k(ps, pd, eidx_flat)


def _mlp_compute(ef, acc, w1e_ref, w2_ref, b1_ref, b2_ref, g_ref, bb_ref):
    h = (jnp.dot(ef, w1e_ref[...], preferred_element_type=jnp.float32)
         + acc + b1_ref[...])
    h = h * jax.nn.sigmoid(h)
    o = jnp.dot(h, w2_ref[...], preferred_element_type=jnp.float32) + b2_ref[...]
    mu = jnp.mean(o, axis=-1, keepdims=True)
    var = jnp.mean((o - mu) ** 2, axis=-1, keepdims=True)
    o = g_ref[...] * (o - mu) * lax.rsqrt(var + 1e-5) + bb_ref[...]
    return o + ef


def _mlp_first_body(ef_ref, acc_ref, w1e_ref, w2_ref, b1_ref, b2_ref, g_ref,
                    bb_ref, out_ref):
    out_ref[...] = _mlp_compute(ef_ref[...], acc_ref[...], w1e_ref, w2_ref,
                                b1_ref, b2_ref, g_ref, bb_ref)


def _mlp_chain_body(carry_ref, ef_ref, acc_ref, w1e_ref, w2_ref, b1_ref,
                    b2_ref, g_ref, bb_ref, out_ref):
    del carry_ref
    out_ref[...] = _mlp_compute(ef_ref[...], acc_ref[...], w1e_ref, w2_ref,
                                b1_ref, b2_ref, g_ref, bb_ref)


def _edge_mlp(ef, accs, w1e, w2, b1r, b2r, gr, br):
    E = ef.shape[0]
    vspec = pl.BlockSpec((1, H), lambda i: (0, 0))
    wspec = [
        pl.BlockSpec((D, H), lambda i: (0, 0)),
        pl.BlockSpec((H, D), lambda i: (0, 0)),
        vspec, vspec, vspec, vspec,
    ]
    out = None
    for acc, base, seg_e in accs:
        nblk = seg_e // _BE
        bb = base // _BE
        ef_spec = pl.BlockSpec((_BE, D), lambda i, bb=bb: (bb + i, 0))
        acc_spec = pl.BlockSpec((_BE, H), lambda i: (i, 0))
        out_spec = pl.BlockSpec((_BE, D), lambda i, bb=bb: (bb + i, 0))
        if out is None:
            out = pl.pallas_call(
                _mlp_first_body,
                grid=(nblk,),
                in_specs=[ef_spec, acc_spec] + wspec,
                out_specs=out_spec,
                out_shape=jax.ShapeDtypeStruct((E, D), jnp.float32),
            )(ef, acc, w1e, w2, b1r, b2r, gr, br)
        else:
            out = pl.pallas_call(
                _mlp_chain_body,
                grid=(nblk,),
                in_specs=[pl.BlockSpec(memory_space=pl.ANY), ef_spec,
                          acc_spec] + wspec,
                out_specs=out_spec,
                out_shape=jax.ShapeDtypeStruct((E, D), jnp.float32),
                input_output_aliases={0: 0},
            )(out, ef, acc, w1e, w2, b1r, b2r, gr, br)
    return out


def kernel(mesh_efeat, world_efeat, nfeat, mesh_edge_index, world_edge_index,
           W1, b1, W2, b2, ln_g, ln_b):
    w1e = W1[:D]
    w1s = W1[D:2 * D]
    w1d = W1[2 * D:]

    ps, pd = _node_proj(nfeat, w1s, w1d)

    m_eidx = mesh_edge_index.reshape(-1)
    w_eidx = world_edge_index.reshape(-1)
    E_M = mesh_edge_index.shape[1]
    E_W = world_edge_index.shape[1]

    def build_segs(eidx, E, segs):
        assert sum(segs) == E
        entries, base = [], 0
        for s in segs:
            entries.append((base, s))
            base += s
        return [(_gather_add(ps, pd, eidx, E, b, s), b, s)
                for b, s in entries]

    acc_m = build_segs(m_eidx, E_M, _MESH_SEGS)
    acc_w = build_segs(w_eidx, E_W, _WORLD_SEGS)

    b1r = b1.reshape(1, H)
    b2r = b2.reshape(1, D)
    gr = ln_g.reshape(1, D)
    br = ln_b.reshape(1, D)

    mesh_new = _edge_mlp(mesh_efeat, acc_m, w1e, W2, b1r, b2r, gr, br)
    world_new = _edge_mlp(world_efeat, acc_w, w1e, W2, b1r, b2r, gr, br)
    return (mesh_new, world_new, nfeat)
